# baseline (speedup 1.0000x reference)
import functools

import jax
import jax.numpy as jnp
from jax import lax
from jax.experimental import pallas as pl
from jax.experimental.pallas import tpu as pltpu
from jax.experimental.pallas import tpu_sc as plsc

_EB = 4


def _make_sc_gather(S, D, N):
    info = plsc.get_sparse_core_info()
    NC, NS = info.num_cores, info.num_subcores
    NW = NC * NS
    b_per_w = N // NW
    mesh = plsc.VectorSubcoreMesh(core_axis_name="c", subcore_axis_name="s")

    @functools.partial(
        pl.kernel, mesh=mesh,
        out_type=jax.ShapeDtypeStruct((N, D), jnp.float32),
        scratch_types=[
            pltpu.VMEM((b_per_w,), jnp.int32),
            pltpu.VMEM((b_per_w, D), jnp.float32),
            pltpu.SemaphoreType.DMA,
        ],
    )
    def sc_gather(table_hbm, idx_hbm, out_hbm, idx_v, rows_v, sem):
        wid = lax.axis_index("s") * NC + lax.axis_index("c")
        base = wid * b_per_w
        pltpu.sync_copy(idx_hbm.at[pl.ds(base, b_per_w)], idx_v)
        pltpu.async_copy(table_hbm.at[idx_v], rows_v, sem).wait()
        pltpu.sync_copy(rows_v, out_hbm.at[pl.ds(base, b_per_w)])

    return sc_gather


def _router_kernel(x_ref, wg_ref, bg_ref, wc1_ref, bc1_ref, wc2_ref, bc2_ref,
                   cap_ref, gate_ref, idx_ref, ones_ref):
    S, D = x_ref.shape
    G = wg_ref.shape[0]
    E = wg_ref.shape[2]
    K = gate_ref.shape[0]
    xf = x_ref[...]

    h = jnp.dot(xf, wc1_ref[...], preferred_element_type=jnp.float32)
    h = h + bc1_ref[...]
    h = h * (1.0 / (1.0 + jnp.exp(-h)))
    cap = jnp.dot(h, wc2_ref[...], preferred_element_type=jnp.float32)
    cap_ref[...] = cap + bc2_ref[...]

    acc = jnp.zeros((S, E), jnp.float32)
    for g in range(G):
        acc = acc + (jnp.dot(xf, wg_ref[g], preferred_element_type=jnp.float32)
                     + bg_ref[g:g + 1, :])
    logits = acc * (1.0 / G)

    mx = jnp.max(logits, axis=1, keepdims=True)
    ex = jnp.exp(logits - mx)
    sc = ex / jnp.sum(ex, axis=1, keepdims=True)

    iota_s = lax.broadcasted_iota(jnp.int32, (S, E), 0)
    iota_k = lax.broadcasted_iota(jnp.int32, (K, E), 0)

    def body(i, carry):
        work, gate, idx = carry
        m = jnp.max(work, axis=0, keepdims=True)
        cand = jnp.where(work == m, iota_s, S)
        sel = jnp.min(cand, axis=0, keepdims=True)
        work = jnp.where(iota_s == sel, -1.0, work)
        gate = jnp.where(iota_k == i, jnp.broadcast_to(m, (K, E)), gate)
        idx = jnp.where(iota_k == i, jnp.broadcast_to(sel, (K, E)), idx)
        return work, gate, idx

    carry = (sc, jnp.zeros((K, E), jnp.float32), jnp.zeros((K, E), jnp.int32))
    for i in range(K):
        carry = body(i, carry)
    work, gate, idx = carry
    gate_ref[...] = gate
    idx_ref[...] = idx
    ones_ref[...] = jnp.where(work < 0.0, 1.0, 0.0)


def _expert_kernel(idx_sref, gate_sref, xg_ref, we_ref, be_ref, out_ref):
    S, D = out_ref.shape
    K = xg_ref.shape[0] // _EB
    eb = pl.program_id(0)

    @pl.when(eb == 0)
    def _():
        out_ref[...] = jnp.zeros_like(out_ref)

    for s in range(_EB):
        e = eb * _EB + s
        y = jnp.dot(xg_ref[s * K:(s + 1) * K, :], we_ref[s],
                    preferred_element_type=jnp.float32)
        y = y + be_ref[s]

        for j in range(K):
            tok = idx_sref[j, e]
            g = gate_sref[j, e]
            out_ref[pl.ds(tok, 1), :] += y[j:j + 1, :] * g


def kernel(x, Wg, bg, Wc1, bc1, Wc2, bc2, We, be):
    B, SEQ, D = x.shape
    G, _, E = Wg.shape
    S = B * SEQ
    K = (S // E)

    xf = x.reshape(S, D)

    cap, gate, idx, ones = pl.pallas_call(
        _router_kernel,
        out_shape=(
            jax.ShapeDtypeStruct((S, E), jnp.float32),
            jax.ShapeDtypeStruct((K, E), jnp.float32),
            jax.ShapeDtypeStruct((K, E), jnp.int32),
            jax.ShapeDtypeStruct((S, E), jnp.float32),
        ),
    )(xf, Wg, bg, Wc1, bc1.reshape(1, D), Wc2, bc2.reshape(1, E))

    idx_flat = idx.T.reshape(E * K)
    xg = _make_sc_gather(S, D, E * K)(xf, idx_flat)

    out = pl.pallas_call(
        _expert_kernel,
        grid_spec=pltpu.PrefetchScalarGridSpec(
            num_scalar_prefetch=2,
            grid=(E // _EB,),
            in_specs=[
                pl.BlockSpec((_EB * K, D), lambda e, *_: (e, 0)),
                pl.BlockSpec((_EB, D, D), lambda e, *_: (e, 0, 0)),
                pl.BlockSpec((_EB, 1, D), lambda e, *_: (e, 0, 0)),
            ],
            out_specs=pl.BlockSpec((S, D), lambda e, *_: (0, 0)),
        ),
        out_shape=jax.ShapeDtypeStruct((S, D), jnp.float32),
        compiler_params=pltpu.CompilerParams(
            dimension_semantics=("arbitrary",),
        ),
    )(idx, gate, xg, We, be.reshape(E, 1, D))

    return (out.reshape(B, SEQ, D), ones.reshape(B, SEQ, E),
            cap.reshape(B, SEQ, E))


# speedup vs baseline: 1.1028x; 1.1028x over previous
import functools

import jax
import jax.numpy as jnp
from jax import lax
from jax.experimental import pallas as pl
from jax.experimental.pallas import tpu as pltpu

_EB = 4


def _router_kernel(x_ref, wg_ref, bg_ref, wc1_ref, bc1_ref, wc2_ref, bc2_ref,
                   cap_ref, gate_ref, idx_ref, ones_ref):
    S, D = x_ref.shape
    G = wg_ref.shape[0]
    E = wg_ref.shape[2]
    K = gate_ref.shape[0]
    xf = x_ref[...]

    h = jnp.dot(xf, wc1_ref[...], preferred_element_type=jnp.float32)
    h = h + bc1_ref[...]
    h = h * (1.0 / (1.0 + jnp.exp(-h)))
    cap = jnp.dot(h, wc2_ref[...], preferred_element_type=jnp.float32)
    cap_ref[...] = cap + bc2_ref[...]

    acc = jnp.zeros((S, E), jnp.float32)
    for g in range(G):
        acc = acc + (jnp.dot(xf, wg_ref[g], preferred_element_type=jnp.float32)
                     + bg_ref[g:g + 1, :])
    logits = acc * (1.0 / G)

    mx = jnp.max(logits, axis=1, keepdims=True)
    ex = jnp.exp(logits - mx)
    sc = ex / jnp.sum(ex, axis=1, keepdims=True)

    iota_k = lax.broadcasted_iota(jnp.int32, (K, E), 0)
    iota_s = lax.broadcasted_iota(jnp.int32, (S, E), 0)

    m = jnp.max(sc, axis=0, keepdims=True)
    sel = jnp.min(jnp.where(sc == m, iota_s, S), axis=0, keepdims=True)
    gate = jnp.broadcast_to(m, (K, E))
    idx = jnp.broadcast_to(sel, (K, E))
    for i in range(1, K):
        elig = (sc < m) | ((sc == m) & (iota_s > sel))
        val = jnp.where(elig, sc, -1.0)
        m = jnp.max(val, axis=0, keepdims=True)
        sel = jnp.min(jnp.where(val == m, iota_s, S), axis=0, keepdims=True)
        gate = jnp.where(iota_k == i, jnp.broadcast_to(m, (K, E)), gate)
        idx = jnp.where(iota_k == i, jnp.broadcast_to(sel, (K, E)), idx)
    gate_ref[...] = gate
    idx_ref[...] = idx
    ones_ref[...] = jnp.where((sc > m) | ((sc == m) & (iota_s <= sel)),
                              1.0, 0.0)


def _expert_kernel(idx_sref, gate_sref, x_ref, we_ref, be_ref, out_ref,
                   rows_ref):
    S, D = x_ref.shape
    K = rows_ref.shape[0]
    eb = pl.program_id(0)

    @pl.when(eb == 0)
    def _():
        out_ref[...] = jnp.zeros_like(out_ref)

    for s in range(_EB):
        e = eb * _EB + s
        for j in range(K):
            tok = idx_sref[j, e]
            rows_ref[j:j + 1, :] = x_ref[pl.ds(tok, 1), :]

        y = jnp.dot(rows_ref[...], we_ref[s],
                    preferred_element_type=jnp.float32)
        y = y + be_ref[s]

        for j in range(K):
            tok = idx_sref[j, e]
            g = gate_sref[j, e]
            out_ref[pl.ds(tok, 1), :] += y[j:j + 1, :] * g


def kernel(x, Wg, bg, Wc1, bc1, Wc2, bc2, We, be):
    B, SEQ, D = x.shape
    G, _, E = Wg.shape
    S = B * SEQ
    K = (S // E)

    xf = x.reshape(S, D)

    cap, gate, idx, ones = pl.pallas_call(
        _router_kernel,
        out_shape=(
            jax.ShapeDtypeStruct((S, E), jnp.float32),
            jax.ShapeDtypeStruct((K, E), jnp.float32),
            jax.ShapeDtypeStruct((K, E), jnp.int32),
            jax.ShapeDtypeStruct((S, E), jnp.float32),
        ),
    )(xf, Wg, bg, Wc1, bc1.reshape(1, D), Wc2, bc2.reshape(1, E))

    out = pl.pallas_call(
        _expert_kernel,
        grid_spec=pltpu.PrefetchScalarGridSpec(
            num_scalar_prefetch=2,
            grid=(E // _EB,),
            in_specs=[
                pl.BlockSpec((S, D), lambda e, *_: (0, 0)),
                pl.BlockSpec((_EB, D, D), lambda e, *_: (e, 0, 0)),
                pl.BlockSpec((_EB, 1, D), lambda e, *_: (e, 0, 0)),
            ],
            out_specs=pl.BlockSpec((S, D), lambda e, *_: (0, 0)),
            scratch_shapes=[pltpu.VMEM((K, D), jnp.float32)],
        ),
        out_shape=jax.ShapeDtypeStruct((S, D), jnp.float32),
        compiler_params=pltpu.CompilerParams(
            dimension_semantics=("arbitrary",),
        ),
    )(idx, gate, xf, We, be.reshape(E, 1, D))

    return (out.reshape(B, SEQ, D), ones.reshape(B, SEQ, E),
            cap.reshape(B, SEQ, E))


# speedup vs baseline: 1.1627x; 1.0543x over previous
import functools

import jax
import jax.numpy as jnp
from jax import lax
from jax.experimental import pallas as pl
from jax.experimental.pallas import tpu as pltpu

_EB = 4


def _router_kernel(x_ref, wg_ref, bg_ref, wc1_ref, bc1_ref, wc2_ref, bc2_ref,
                   cap_ref, gate_ref, idx_ref, ones_ref):
    S, D = x_ref.shape
    G = wg_ref.shape[0]
    E = wg_ref.shape[2]
    K = gate_ref.shape[0]
    xf = x_ref[...]

    h = jnp.dot(xf, wc1_ref[...], preferred_element_type=jnp.float32)
    h = h + bc1_ref[...]
    h = h * (1.0 / (1.0 + jnp.exp(-h)))
    cap = jnp.dot(h, wc2_ref[...], preferred_element_type=jnp.float32)
    cap_ref[...] = cap + bc2_ref[...]

    acc = jnp.zeros((S, E), jnp.float32)
    for g in range(G):
        acc = acc + (jnp.dot(xf, wg_ref[g], preferred_element_type=jnp.float32)
                     + bg_ref[g:g + 1, :])
    logits = acc * (1.0 / G)

    mx = jnp.max(logits, axis=1, keepdims=True)
    ex = jnp.exp(logits - mx)
    sc = ex / jnp.sum(ex, axis=1, keepdims=True)

    iota_s = lax.broadcasted_iota(jnp.int32, (S, E), 0)
    iota_k = lax.broadcasted_iota(jnp.int32, (K, E), 0)

    def body(i, carry):
        work, gate, idx = carry
        m = jnp.max(work, axis=0, keepdims=True)
        cand = jnp.where(work == m, iota_s, S)
        sel = jnp.min(cand, axis=0, keepdims=True)
        work = jnp.where(iota_s == sel, -1.0, work)
        gate = jnp.where(iota_k == i, jnp.broadcast_to(m, (K, E)), gate)
        idx = jnp.where(iota_k == i, jnp.broadcast_to(sel, (K, E)), idx)
        return work, gate, idx

    carry = (sc, jnp.zeros((K, E), jnp.float32), jnp.zeros((K, E), jnp.int32))
    for i in range(K):
        carry = body(i, carry)
    work, gate, idx = carry
    gate_ref[...] = gate
    idx_ref[...] = idx
    ones_ref[...] = jnp.where(work < 0.0, 1.0, 0.0)


def _expert_kernel(idx_sref, gate_sref, x_ref, we_ref, be_ref, out_ref,
                   rows_ref):
    S, D = x_ref.shape
    K = rows_ref.shape[0]
    eb = pl.program_id(0)

    @pl.when(eb == 0)
    def _():
        out_ref[...] = jnp.zeros_like(out_ref)

    for s in range(_EB):
        e = eb * _EB + s
        for j in range(K):
            tok = idx_sref[j, e]
            rows_ref[j:j + 1, :] = x_ref[pl.ds(tok, 1), :]

        y = jnp.dot(rows_ref[...], we_ref[s],
                    preferred_element_type=jnp.float32)
        y = y + be_ref[s]

        for j in range(K):
            tok = idx_sref[j, e]
            g = gate_sref[j, e]
            out_ref[pl.ds(tok, 1), :] += y[j:j + 1, :] * g


def kernel(x, Wg, bg, Wc1, bc1, Wc2, bc2, We, be):
    B, SEQ, D = x.shape
    G, _, E = Wg.shape
    S = B * SEQ
    K = (S // E)

    xf = x.reshape(S, D)

    cap, gate, idx, ones = pl.pallas_call(
        _router_kernel,
        out_shape=(
            jax.ShapeDtypeStruct((S, E), jnp.float32),
            jax.ShapeDtypeStruct((K, E), jnp.float32),
            jax.ShapeDtypeStruct((K, E), jnp.int32),
            jax.ShapeDtypeStruct((S, E), jnp.float32),
        ),
    )(xf, Wg, bg, Wc1, bc1.reshape(1, D), Wc2, bc2.reshape(1, E))

    out = pl.pallas_call(
        _expert_kernel,
        grid_spec=pltpu.PrefetchScalarGridSpec(
            num_scalar_prefetch=2,
            grid=(E // _EB,),
            in_specs=[
                pl.BlockSpec((S, D), lambda e, *_: (0, 0)),
                pl.BlockSpec((_EB, D, D), lambda e, *_: (e, 0, 0)),
                pl.BlockSpec((_EB, 1, D), lambda e, *_: (e, 0, 0)),
            ],
            out_specs=pl.BlockSpec((S, D), lambda e, *_: (0, 0)),
            scratch_shapes=[pltpu.VMEM((K, D), jnp.float32)],
        ),
        out_shape=jax.ShapeDtypeStruct((S, D), jnp.float32),
        compiler_params=pltpu.CompilerParams(
            dimension_semantics=("arbitrary",),
        ),
    )(idx, gate, xf, We, be.reshape(E, 1, D))

    return (out.reshape(B, SEQ, D), ones.reshape(B, SEQ, E),
            cap.reshape(B, SEQ, E))
